# baseline (speedup 1.0000x reference)
import functools

import jax
import jax.numpy as jnp
from jax.experimental import pallas as pl

_NEG = float("-inf")


def _mega_body(lg_ref, *refs, n_kv, rows, cols, beam):
    kv_in = refs[:n_kv]
    probs_ref = refs[n_kv]
    idx_ref = refs[n_kv + 1]
    kv_out = refs[n_kv + 2:]

    for i in range(n_kv):
        kv_out[i][...] = kv_in[i][...][None]

    i0 = pl.program_id(0)
    b0 = pl.program_id(1)

    @pl.when((i0 == 0) & (b0 == 0))
    def _topk_lse():
        X0 = lg_ref[...]
        flat = (jax.lax.broadcasted_iota(jnp.int32, (rows, cols), 0) * cols
                + jax.lax.broadcasted_iota(jnp.int32, (rows, cols), 1))
        big = jnp.int32(2**30)

        X = X0
        vals, gsel = [], []
        for k in range(beam):
            m = jnp.max(X)
            g = jnp.min(jnp.where(X == m, flat, big))
            vals.append(m)
            gsel.append(g)
            if k + 1 < beam:
                X = jnp.where(flat == g, _NEG, X)

        m0 = vals[0]
        lse = jnp.log(jnp.sum(jnp.exp(X0 - m0))) + m0

        lane_b = jax.lax.broadcasted_iota(jnp.int32, (1, beam), 1)
        pv = jnp.zeros((1, beam), jnp.float32)
        iv = jnp.zeros((1, beam), jnp.int32)
        for k in range(beam):
            pv = jnp.where(lane_b == k, vals[k] - lse, pv)
            iv = jnp.where(lane_b == k, gsel[k], iv)
        probs_ref[...] = pv
        idx_ref[...] = iv


def kernel(kv_0, kv_1, kv_2, kv_3, kv_4, kv_5, kv_6, kv_7, kv_8, kv_9,
           kv_10, kv_11, logits, save_id, beam_size):
    kvs = [kv_0, kv_1, kv_2, kv_3, kv_4, kv_5, kv_6, kv_7, kv_8, kv_9,
           kv_10, kv_11]
    n_kv = len(kvs)
    beam = save_id.shape[0]
    kv_shape = kvs[0].shape
    flat = kv_shape[2] * kv_shape[3]

    vocab = logits.shape[-1]
    rows = 8
    cols = vocab // rows
    lg = logits.reshape(rows, cols)
    kv2 = [kv.reshape(kv_shape[1], flat) for kv in kvs]

    ch = 8192
    ns = flat // ch

    body = functools.partial(_mega_body, n_kv=n_kv, rows=rows, cols=cols,
                             beam=beam)
    in_specs = [pl.BlockSpec((rows, cols), lambda i, b: (0, 0))]
    in_specs += [pl.BlockSpec((kv_shape[1], ch), lambda i, b: (0, i))] * n_kv
    out_specs = [pl.BlockSpec((1, beam), lambda i, b: (0, 0)),
                 pl.BlockSpec((1, beam), lambda i, b: (0, 0))]
    out_specs += [pl.BlockSpec((1, kv_shape[1], ch),
                               lambda i, b: (b, 0, i))] * n_kv
    out_shape = [jax.ShapeDtypeStruct((1, beam), jnp.float32),
                 jax.ShapeDtypeStruct((1, beam), jnp.int32)]
    out_shape += [jax.ShapeDtypeStruct((beam, kv_shape[1], flat),
                                       jnp.float32)] * n_kv

    outs = pl.pallas_call(
        body,
        grid=(ns, beam),
        in_specs=in_specs,
        out_specs=out_specs,
        out_shape=out_shape,
    )(lg, *kv2)

    probs, idx = outs[0], outs[1]
    kv_outs = [o.reshape((beam,) + kv_shape[1:]) for o in outs[2:]]

    idx_t = idx.reshape(beam, 1)
    save_id_out = jnp.concatenate([save_id, idx_t], axis=-1)
    probs_t = probs.reshape(beam, 1)
    bz = jnp.asarray(beam_size, jnp.int32) - jnp.int32(beam)
    max_idx = idx_t[0] + bz
    return (*kv_outs, idx_t, save_id_out, probs_t, max_idx)


# speedup vs baseline: 1.0219x; 1.0219x over previous
import functools

import jax
import jax.numpy as jnp
from jax.experimental import pallas as pl
from jax.experimental.pallas import tpu as pltpu

_NEG = float("-inf")


def _mega_body(lg_hbm, *refs, n_kv, rows, cols, beam, ns):
    kv_in = refs[:n_kv]
    probs_ref = refs[n_kv]
    idx_ref = refs[n_kv + 1]
    kv_out = refs[n_kv + 2:n_kv + 2 + n_kv]
    lg_vmem, sem = refs[n_kv + 2 + n_kv:]

    i0 = pl.program_id(0)

    @pl.when(i0 == 0)
    def _start_logits_fetch():
        pltpu.make_async_copy(lg_hbm, lg_vmem, sem).start()

    for i in range(n_kv):
        x = kv_in[i][...]
        kv_out[i][...] = jnp.broadcast_to(x[None], (beam,) + x.shape)

    @pl.when(i0 == min(2, ns - 1))
    def _topk_lse():
        pltpu.make_async_copy(lg_hbm, lg_vmem, sem).wait()
        X0 = lg_vmem[...]
        flat = (jax.lax.broadcasted_iota(jnp.int32, (rows, cols), 0) * cols
                + jax.lax.broadcasted_iota(jnp.int32, (rows, cols), 1))
        big = jnp.int32(2**30)

        X = X0
        vals, gsel = [], []
        for k in range(beam):
            m = jnp.max(X)
            g = jnp.min(jnp.where(X == m, flat, big))
            vals.append(m)
            gsel.append(g)
            if k + 1 < beam:
                X = jnp.where(flat == g, _NEG, X)

        m0 = vals[0]
        lse = jnp.log(jnp.sum(jnp.exp(X0 - m0))) + m0

        lane_b = jax.lax.broadcasted_iota(jnp.int32, (1, beam), 1)
        pv = jnp.zeros((1, beam), jnp.float32)
        iv = jnp.zeros((1, beam), jnp.int32)
        for k in range(beam):
            pv = jnp.where(lane_b == k, vals[k] - lse, pv)
            iv = jnp.where(lane_b == k, gsel[k], iv)
        probs_ref[...] = pv
        idx_ref[...] = iv


def kernel(kv_0, kv_1, kv_2, kv_3, kv_4, kv_5, kv_6, kv_7, kv_8, kv_9,
           kv_10, kv_11, logits, save_id, beam_size):
    kvs = [kv_0, kv_1, kv_2, kv_3, kv_4, kv_5, kv_6, kv_7, kv_8, kv_9,
           kv_10, kv_11]
    n_kv = len(kvs)
    beam = save_id.shape[0]
    kv_shape = kvs[0].shape
    flat = kv_shape[2] * kv_shape[3]

    vocab = logits.shape[-1]
    rows = 8
    cols = vocab // rows
    lg = logits.reshape(rows, cols)
    kv2 = [kv.reshape(kv_shape[1], flat) for kv in kvs]

    ch = 4096
    ns = flat // ch

    body = functools.partial(_mega_body, n_kv=n_kv, rows=rows, cols=cols,
                             beam=beam, ns=ns)
    in_specs = [pl.BlockSpec(memory_space=pl.ANY)]
    in_specs += [pl.BlockSpec((kv_shape[1], ch), lambda i: (0, i))] * n_kv
    out_specs = [pl.BlockSpec((1, beam), lambda i: (0, 0)),
                 pl.BlockSpec((1, beam), lambda i: (0, 0))]
    out_specs += [pl.BlockSpec((beam, kv_shape[1], ch),
                               lambda i: (0, 0, i))] * n_kv
    out_shape = [jax.ShapeDtypeStruct((1, beam), jnp.float32),
                 jax.ShapeDtypeStruct((1, beam), jnp.int32)]
    out_shape += [jax.ShapeDtypeStruct((beam, kv_shape[1], flat),
                                       jnp.float32)] * n_kv

    outs = pl.pallas_call(
        body,
        grid=(ns,),
        in_specs=in_specs,
        out_specs=out_specs,
        out_shape=out_shape,
        scratch_shapes=[pltpu.VMEM((rows, cols), jnp.float32),
                        pltpu.SemaphoreType.DMA],
    )(lg, *kv2)

    probs, idx = outs[0], outs[1]
    kv_outs = [o.reshape((beam,) + kv_shape[1:]) for o in outs[2:]]

    idx_t = idx.reshape(beam, 1)
    save_id_out = jnp.concatenate([save_id, idx_t], axis=-1)
    probs_t = probs.reshape(beam, 1)
    bz = jnp.asarray(beam_size, jnp.int32) - jnp.int32(beam)
    max_idx = idx_t[0] + bz
    return (*kv_outs, idx_t, save_id_out, probs_t, max_idx)


# speedup vs baseline: 1.4235x; 1.3930x over previous
import functools

import jax
import jax.numpy as jnp
from jax.experimental import pallas as pl
from jax.experimental.pallas import tpu as pltpu

_NEG = float("-inf")


def _mega_body(lg_ref, *refs, n_kv, cl, n_lg, vocab, beam, ns, blk):
    kv_in = refs[:n_kv]
    probs_ref = refs[n_kv]
    idx_ref = refs[n_kv + 1]
    kv_out = refs[n_kv + 2:n_kv + 2 + n_kv]
    runv_ref, runi_ref, mrun_ref, srun_ref = refs[n_kv + 2 + n_kv:]

    i0 = pl.program_id(0)

    for i in range(n_kv):
        kv_out[i][...] = jnp.broadcast_to(kv_in[i][...], (beam,) + blk[1:])

    big = jnp.int32(2**30)

    @pl.when(i0 == 0)
    def _init():
        runv_ref[...] = jnp.full((1, beam), _NEG, jnp.float32)
        runi_ref[...] = jnp.full((1, beam), big, jnp.int32)
        mrun_ref[...] = jnp.full((1, 128), _NEG, jnp.float32)
        srun_ref[...] = jnp.zeros((1, 128), jnp.float32)

    @pl.when(i0 < n_lg)
    def _chunk():
        lane = jax.lax.broadcasted_iota(jnp.int32, (1, cl), 1)
        vi = lane + i0 * cl
        x = jnp.where(vi < vocab, lg_ref[...], _NEG)

        cv, ci = [], []
        xm = x
        for k in range(beam):
            m = jnp.max(xm)
            g = jnp.min(jnp.where(xm == m, vi, big))
            cv.append(m)
            ci.append(g)
            if k + 1 < beam:
                xm = jnp.where(vi == g, _NEG, xm)

        lane2 = jax.lax.broadcasted_iota(jnp.int32, (1, 2 * beam), 1)
        mv = runv_ref[...]
        mi = runi_ref[...]
        candv = jnp.concatenate([mv, jnp.zeros((1, beam), jnp.float32)], -1)
        candi = jnp.concatenate([mi, jnp.full((1, beam), big, jnp.int32)], -1)
        for k in range(beam):
            candv = jnp.where(lane2 == beam + k, cv[k], candv)
            candi = jnp.where(lane2 == beam + k, ci[k], candi)
        lane_b = jax.lax.broadcasted_iota(jnp.int32, (1, beam), 1)
        nv = jnp.zeros((1, beam), jnp.float32)
        ni = jnp.zeros((1, beam), jnp.int32)
        for k in range(beam):
            m = jnp.max(candv)
            g = jnp.min(jnp.where(candv == m, candi, big))
            nv = jnp.where(lane_b == k, m, nv)
            ni = jnp.where(lane_b == k, g, ni)
            candv = jnp.where(candi == g, _NEG, candv)
        runv_ref[...] = nv
        runi_ref[...] = ni

        cm = cv[0]
        mold = mrun_ref[...]
        mnew = jnp.maximum(mold, cm)
        s_chunk = jnp.sum(jnp.exp(x - mnew[0, 0]))
        srun_ref[...] = srun_ref[...] * jnp.exp(mold - mnew) + s_chunk
        mrun_ref[...] = mnew

    @pl.when(i0 == ns - 1)
    def _finalize():
        lse = jnp.log(srun_ref[...]) + mrun_ref[...]
        probs_ref[...] = runv_ref[...] - lse[0, 0]
        idx_ref[...] = runi_ref[...]


def kernel(kv_0, kv_1, kv_2, kv_3, kv_4, kv_5, kv_6, kv_7, kv_8, kv_9,
           kv_10, kv_11, logits, save_id, beam_size):
    kvs = [kv_0, kv_1, kv_2, kv_3, kv_4, kv_5, kv_6, kv_7, kv_8, kv_9,
           kv_10, kv_11]
    n_kv = len(kvs)
    beam = save_id.shape[0]
    kv_shape = kvs[0].shape
    vocab = logits.shape[-1]

    cs = 32
    ns = kv_shape[2] // cs
    blk = (1, kv_shape[1], cs, kv_shape[3])

    cl = 16384
    n_lg = -(-vocab // cl)
    assert n_lg <= ns
    lg_last = n_lg - 1

    body = functools.partial(_mega_body, n_kv=n_kv, cl=cl, n_lg=n_lg,
                             vocab=vocab, beam=beam, ns=ns, blk=blk)
    in_specs = [pl.BlockSpec((1, cl),
                             lambda i: (0, jnp.minimum(i, lg_last)))]
    in_specs += [pl.BlockSpec(blk, lambda i: (0, 0, i, 0))] * n_kv
    out_specs = [pl.BlockSpec((1, beam), lambda i: (0, 0)),
                 pl.BlockSpec((1, beam), lambda i: (0, 0))]
    out_specs += [pl.BlockSpec((beam,) + blk[1:],
                               lambda i: (0, 0, i, 0))] * n_kv
    out_shape = [jax.ShapeDtypeStruct((1, beam), jnp.float32),
                 jax.ShapeDtypeStruct((1, beam), jnp.int32)]
    out_shape += [jax.ShapeDtypeStruct((beam,) + kv_shape[1:],
                                       jnp.float32)] * n_kv

    outs = pl.pallas_call(
        body,
        grid=(ns,),
        in_specs=in_specs,
        out_specs=out_specs,
        out_shape=out_shape,
        scratch_shapes=[pltpu.VMEM((1, beam), jnp.float32),
                        pltpu.VMEM((1, beam), jnp.int32),
                        pltpu.VMEM((1, 128), jnp.float32),
                        pltpu.VMEM((1, 128), jnp.float32)],
    )(logits, *kvs)

    probs, idx = outs[0], outs[1]
    kv_outs = list(outs[2:])

    idx_t = idx.reshape(beam, 1)
    save_id_out = jnp.concatenate([save_id, idx_t], axis=-1)
    probs_t = probs.reshape(beam, 1)
    bz = jnp.asarray(beam_size, jnp.int32) - jnp.int32(beam)
    max_idx = idx_t[0] + bz
    return (*kv_outs, idx_t, save_id_out, probs_t, max_idx)


# speedup vs baseline: 1.4664x; 1.0301x over previous
import functools

import jax
import jax.numpy as jnp
from jax.experimental import pallas as pl
from jax.experimental.pallas import tpu as pltpu

_NEG = float("-inf")


def _chunk_top(x, vi, beam):
    big = jnp.int32(2**30)
    cv, ci = [], []
    for k in range(beam):
        m = jnp.max(x)
        g = jnp.min(jnp.where(x == m, vi, big))
        cv.append(m)
        ci.append(g)
        if k + 1 < beam:
            x = jnp.where(vi == g, _NEG, x)
    return cv, ci


def _body(lg_ref, *refs, n_kv, vocab, beam, cl, n_slots):
    kv_in = refs[:n_kv]
    probs_ref = refs[n_kv]
    idx_ref = refs[n_kv + 1]
    kv_out = refs[n_kv + 2:n_kv + 2 + n_kv]
    slots, in_sems, out_sems = refs[n_kv + 2 + n_kv:]

    def in_copy(j):
        return pltpu.make_async_copy(kv_in[j].at[0], slots.at[j % n_slots],
                                     in_sems.at[j % n_slots])

    def out_copy(j, b):
        return pltpu.make_async_copy(slots.at[j % n_slots], kv_out[j].at[b],
                                     out_sems.at[j % n_slots, b])

    cand_v, cand_i = [], []
    cms, css = [], []

    for j in range(min(n_slots - 1, n_kv)):
        in_copy(j).start()

    for j in range(n_kv):
        in_copy(j).wait()
        for b in range(beam):
            out_copy(j, b).start()
        nxt = j + n_slots - 1
        if nxt < n_kv:
            if nxt - n_slots >= 0:
                for b in range(beam):
                    out_copy(nxt - n_slots, b).wait()
            in_copy(nxt).start()

        off = j * cl
        size = min(cl, vocab - off)
        x = lg_ref[:, pl.ds(off, size)]
        vi = jax.lax.broadcasted_iota(jnp.int32, (1, size), 1) + off
        cv, ci = _chunk_top(x, vi, beam)
        cand_v.extend(cv)
        cand_i.extend(ci)
        cms.append(cv[0])
        css.append(jnp.sum(jnp.exp(x - cv[0])))

    for j in range(max(0, n_kv - n_slots), n_kv):
        for b in range(beam):
            out_copy(j, b).wait()

    big = jnp.int32(2**30)
    nc = len(cand_v)
    lane_c = jax.lax.broadcasted_iota(jnp.int32, (1, nc), 1)
    candv = jnp.zeros((1, nc), jnp.float32)
    candi = jnp.zeros((1, nc), jnp.int32)
    for k in range(nc):
        candv = jnp.where(lane_c == k, cand_v[k], candv)
        candi = jnp.where(lane_c == k, cand_i[k], candi)

    mg = cms[0]
    for c in cms[1:]:
        mg = jnp.maximum(mg, c)
    sg = css[0] * jnp.exp(cms[0] - mg)
    for c, s in zip(cms[1:], css[1:]):
        sg = sg + s * jnp.exp(c - mg)
    lse = jnp.log(sg) + mg

    lane_b = jax.lax.broadcasted_iota(jnp.int32, (1, beam), 1)
    pv = jnp.zeros((1, beam), jnp.float32)
    iv = jnp.zeros((1, beam), jnp.int32)
    for k in range(beam):
        m = jnp.max(candv)
        g = jnp.min(jnp.where(candv == m, candi, big))
        pv = jnp.where(lane_b == k, m - lse, pv)
        iv = jnp.where(lane_b == k, g, iv)
        candv = jnp.where(candi == g, _NEG, candv)
    probs_ref[...] = pv
    idx_ref[...] = iv


def kernel(kv_0, kv_1, kv_2, kv_3, kv_4, kv_5, kv_6, kv_7, kv_8, kv_9,
           kv_10, kv_11, logits, save_id, beam_size):
    kvs = [kv_0, kv_1, kv_2, kv_3, kv_4, kv_5, kv_6, kv_7, kv_8, kv_9,
           kv_10, kv_11]
    n_kv = len(kvs)
    beam = save_id.shape[0]
    kv_shape = kvs[0].shape
    vocab = logits.shape[-1]

    cl = 83456
    assert (n_kv - 1) * cl < vocab <= n_kv * cl
    n_slots = 2

    body = functools.partial(_body, n_kv=n_kv, vocab=vocab, beam=beam,
                             cl=cl, n_slots=n_slots)
    in_specs = [pl.BlockSpec(memory_space=pltpu.MemorySpace.VMEM)]
    in_specs += [pl.BlockSpec(memory_space=pl.ANY)] * n_kv
    out_specs = [pl.BlockSpec(memory_space=pltpu.MemorySpace.VMEM),
                 pl.BlockSpec(memory_space=pltpu.MemorySpace.VMEM)]
    out_specs += [pl.BlockSpec(memory_space=pl.ANY)] * n_kv
    out_shape = [jax.ShapeDtypeStruct((1, beam), jnp.float32),
                 jax.ShapeDtypeStruct((1, beam), jnp.int32)]
    out_shape += [jax.ShapeDtypeStruct((beam,) + kv_shape[1:],
                                       jnp.float32)] * n_kv

    outs = pl.pallas_call(
        body,
        in_specs=in_specs,
        out_specs=out_specs,
        out_shape=out_shape,
        scratch_shapes=[pltpu.VMEM((n_slots,) + kv_shape[1:], jnp.float32),
                        pltpu.SemaphoreType.DMA((n_slots,)),
                        pltpu.SemaphoreType.DMA((n_slots, beam))],
    )(logits, *kvs)

    probs, idx = outs[0], outs[1]
    kv_outs = list(outs[2:])

    idx_t = idx.reshape(beam, 1)
    save_id_out = jnp.concatenate([save_id, idx_t], axis=-1)
    probs_t = probs.reshape(beam, 1)
    bz = jnp.asarray(beam_size, jnp.int32) - jnp.int32(beam)
    max_idx = idx_t[0] + bz
    return (*kv_outs, idx_t, save_id_out, probs_t, max_idx)
